# Initial kernel scaffold; baseline (speedup 1.0000x reference)
import jax
import jax.numpy as jnp
from jax.experimental import pallas as pl


def kernel(words, bigram, emb, emb_bigram, W1, b1, W2, b2):
    raise NotImplementedError("write your pallas kernel here")


# speedup vs baseline: 4.6314x; 4.6314x over previous
import functools

import jax
import jax.numpy as jnp
from jax import lax
from jax.experimental import pallas as pl
from jax.experimental.pallas import tpu as pltpu
from jax.experimental.pallas import tpu_sc as plsc

B, L = 4096, 200
D = 64
HIDDEN = 256
NUM_CLASSES = 10

NC, NS = 2, 16
NW = NC * NS
BPW = B // NW
CH = 100
NCH = L // CH
HALF = BPW // 2

_mesh = plsc.VectorSubcoreMesh(core_axis_name="c", subcore_axis_name="s")


@functools.partial(
    pl.kernel,
    out_type=jax.ShapeDtypeStruct((B, 2 * D), jnp.float32),
    mesh=_mesh,
    scratch_types=[
        pltpu.VMEM((NCH * BPW, CH), jnp.int32),
        pltpu.VMEM((2, L, D), jnp.float32),
        pltpu.VMEM((BPW, 2 * D), jnp.float32),
        pltpu.SemaphoreType.DMA,
        pltpu.SemaphoreType.DMA,
    ],
    compiler_params=pltpu.CompilerParams(use_tc_tiling_on_sc=False),
)
def _pool(words_hbm, bigram_hbm, emb_hbm, embb_hbm, out_hbm,
          idx_v, buf_v, out_v, sem0, sem1):
    wid = lax.axis_index("c") * NS + lax.axis_index("s")
    base = wid * BPW

    def phase(table_hbm, idx_hbm, col):
        pltpu.sync_copy(idx_hbm.at[pl.ds(NCH * base, NCH * BPW)], idx_v)

        def issue(r, slot, sem):
            for c in range(NCH):
                pltpu.async_copy(
                    table_hbm.at[idx_v.at[NCH * r + c]],
                    buf_v.at[slot, pl.ds(c * CH, CH)],
                    sem)

        def drain(r, slot, sem):
            for c in range(NCH):
                pltpu.make_async_copy(
                    table_hbm.at[idx_v.at[NCH * r + c]],
                    buf_v.at[slot, pl.ds(c * CH, CH)],
                    sem).wait()

        def reduce(r, slot):
            def rbody(j, accs):
                new = list(accs)
                for k in range(4):
                    row = 4 * j + k
                    for d in range(4):
                        new[d] = new[d] + buf_v[slot, row, pl.ds(16 * d, 16)]
                return tuple(new)
            z = jnp.zeros((16,), jnp.float32)
            accs = lax.fori_loop(0, L // 4, rbody, (z, z, z, z))
            for d in range(4):
                out_v[r, pl.ds(col + 16 * d, 16)] = accs[d] * jnp.float32(1.0 / L)

        issue(0, 0, sem0)
        issue(1, 1, sem1)

        def body(r2, carry):
            r0 = 2 * r2
            drain(r0, 0, sem0)
            reduce(r0, 0)

            @pl.when(r2 < HALF - 1)
            def _():
                issue(r0 + 2, 0, sem0)

            drain(r0 + 1, 1, sem1)
            reduce(r0 + 1, 1)

            @pl.when(r2 < HALF - 1)
            def _():
                issue(r0 + 3, 1, sem1)

            return carry

        lax.fori_loop(0, HALF, body, 0)

    phase(emb_hbm, words_hbm, 0)
    phase(embb_hbm, bigram_hbm, D)

    pltpu.sync_copy(out_v, out_hbm.at[pl.ds(base, BPW)])


def _mlp_body(x_ref, w1_ref, b1_ref, w2_ref, b2_ref, o_ref):
    h = jnp.dot(x_ref[...], w1_ref[...], preferred_element_type=jnp.float32)
    h = jnp.maximum(h + b1_ref[...], 0.0)
    o = jnp.dot(h, w2_ref[...], preferred_element_type=jnp.float32)
    o_ref[...] = o + b2_ref[...]


_BM = 512


def _mlp(pooled, w1t, b1r, w2p, b2p):
    return pl.pallas_call(
        _mlp_body,
        grid=(B // _BM,),
        in_specs=[
            pl.BlockSpec((_BM, 2 * D), lambda i: (i, 0)),
            pl.BlockSpec((2 * D, HIDDEN), lambda i: (0, 0)),
            pl.BlockSpec((1, HIDDEN), lambda i: (0, 0)),
            pl.BlockSpec((HIDDEN, 128), lambda i: (0, 0)),
            pl.BlockSpec((1, 128), lambda i: (0, 0)),
        ],
        out_specs=pl.BlockSpec((_BM, 128), lambda i: (i, 0)),
        out_shape=jax.ShapeDtypeStruct((B, 128), jnp.float32),
    )(pooled, w1t, b1r, w2p, b2p)


def kernel(words, bigram, emb, emb_bigram, W1, b1, W2, b2):
    words2 = words.reshape(NCH * B, CH)
    bigram2 = bigram.reshape(NCH * B, CH)
    pooled = _pool(words2, bigram2, emb, emb_bigram)

    w1t = W1.T
    b1r = b1.reshape(1, HIDDEN)
    w2p = jnp.zeros((HIDDEN, 128), W2.dtype).at[:, :NUM_CLASSES].set(W2.T)
    b2p = jnp.zeros((1, 128), b2.dtype).at[0, :NUM_CLASSES].set(b2)
    out = _mlp(pooled, w1t, b1r, w2p, b2p)
    return out[:, :NUM_CLASSES]
